# Initial kernel scaffold; baseline (speedup 1.0000x reference)
import jax
import jax.numpy as jnp
from jax.experimental import pallas as pl


def kernel(x, gate_w, gate_b, W1, B1, W2, B2):
    raise NotImplementedError("write your pallas kernel here")


# speedup vs baseline: 2.0428x; 2.0428x over previous
import functools
import jax
import jax.numpy as jnp
import numpy as np
from jax.experimental import pallas as pl
from jax.experimental.pallas import tpu as pltpu

BM = 256
BF = 768
NEXP = 8
EPAD = 128


def _router_kernel(x_ref, gw_ref, gb_ref, e0_ref, e1_ref, w0_ref, w1_ref, us_ref):
    logits = jnp.dot(x_ref[...], gw_ref[...], preferred_element_type=jnp.float32)
    logits = logits + gb_ref[0][None, :]
    col = jax.lax.broadcasted_iota(jnp.int32, logits.shape, 1)
    big = jnp.int32(2 ** 30)
    m1 = jnp.max(logits, axis=1)
    a1 = jnp.min(jnp.where(logits == m1[:, None], col, big), axis=1)
    l2 = jnp.where(col == a1[:, None], -1e30, logits)
    m2 = jnp.max(l2, axis=1)
    a2 = jnp.min(jnp.where(l2 == m2[:, None], col, big), axis=1)
    r = jnp.exp(m2 - m1)
    z = 1.0 / (1.0 + r)
    e0_ref[...] = a1
    e1_ref[...] = a2
    w0_ref[...] = z
    w1_ref[...] = r * z
    p = jnp.exp(logits - m1[:, None])
    p = p / jnp.sum(p, axis=1, keepdims=True)
    us_ref[...] = jnp.sum(p, axis=0)


def _ffn_kernel(be_ref, xs_ref, w1_ref, b1_ref, w2_ref, b2_ref, sw_ref,
                out_ref, acc_ref, *, n_ff_blocks):
    k = pl.program_id(1)

    @pl.when(k == 0)
    def _():
        acc_ref[...] = jnp.zeros_like(acc_ref)

    h = jnp.dot(xs_ref[...], w1_ref[0], preferred_element_type=jnp.float32)
    h = h + b1_ref[0, 0][None, :]
    h = 0.5 * h * (1.0 + jax.lax.erf(h * np.float32(1.0 / np.sqrt(2.0))))
    acc_ref[...] += jnp.dot(h, w2_ref[0], preferred_element_type=jnp.float32)

    @pl.when(k == n_ff_blocks - 1)
    def _():
        out_ref[...] = (acc_ref[...] + b2_ref[0, 0][None, :]) * sw_ref[0, 0][:, None]


def _routing_metadata(e0, e1, w0, w1, S):
    e_flat = jnp.stack([e0, e1], axis=1).reshape(-1)
    w_flat = jnp.stack([w0, w1], axis=1).reshape(-1)
    tok = jnp.arange(2 * S, dtype=jnp.int32) // 2
    onehot = (e_flat[:, None] == jnp.arange(NEXP, dtype=jnp.int32)[None, :])
    onehot = onehot.astype(jnp.int32)
    counts = jnp.sum(onehot, axis=0)
    rank = jnp.cumsum(onehot, axis=0) - onehot
    rank = jnp.sum(rank * onehot, axis=1)
    padded = ((counts + BM - 1) // BM) * BM
    cum_padded = jnp.cumsum(padded)
    offs = cum_padded - padded
    posn = offs[e_flat] + rank
    P = 2 * S + NEXP * BM
    NB = P // BM
    src = jnp.zeros((P,), jnp.int32).at[posn].set(tok)
    sw = jnp.zeros((P,), jnp.float32).at[posn].set(w_flat)
    block_start = jnp.arange(NB, dtype=jnp.int32) * BM
    block_expert = jnp.minimum(
        jnp.searchsorted(cum_padded, block_start, side='right'), NEXP - 1
    ).astype(jnp.int32)
    return src, sw, posn, block_expert, P, NB


def kernel(x, gate_w, gate_b, W1, B1, W2, B2):
    B, S, D = x.shape
    DF = W1.shape[2]
    x2d = x.reshape(B * S, D)
    S = B * S

    gwp = jnp.zeros((D, EPAD), jnp.float32).at[:, :NEXP].set(gate_w)
    gbp = jnp.full((1, EPAD), -1e30, jnp.float32).at[0, :NEXP].set(gate_b)
    e0, e1, w0, w1, usage_sum = pl.pallas_call(
        _router_kernel,
        out_shape=(
            jax.ShapeDtypeStruct((S,), jnp.int32),
            jax.ShapeDtypeStruct((S,), jnp.int32),
            jax.ShapeDtypeStruct((S,), jnp.float32),
            jax.ShapeDtypeStruct((S,), jnp.float32),
            jax.ShapeDtypeStruct((EPAD,), jnp.float32),
        ),
    )(x2d, gwp, gbp)

    usage = usage_sum[:NEXP] / jnp.float32(S)
    loss = NEXP * jnp.sum(usage ** 2) - 1.0

    src, sw, posn, block_expert, P, NB = _routing_metadata(e0, e1, w0, w1, S)

    xs = jnp.take(x2d, src, axis=0)

    KF = DF // BF
    sw3 = sw.reshape(NB, 1, BM)
    B1r = B1.reshape(NEXP * KF, 1, BF)
    B2r = B2.reshape(NEXP, 1, D)
    grid_spec = pltpu.PrefetchScalarGridSpec(
        num_scalar_prefetch=1,
        grid=(NB, KF),
        in_specs=[
            pl.BlockSpec((BM, D), lambda b, k, be: (b, 0)),
            pl.BlockSpec((1, D, BF), lambda b, k, be: (be[b], 0, k)),
            pl.BlockSpec((1, 1, BF), lambda b, k, be, kf=KF: (be[b] * kf + k, 0, 0)),
            pl.BlockSpec((1, BF, D), lambda b, k, be: (be[b], k, 0)),
            pl.BlockSpec((1, 1, D), lambda b, k, be: (be[b], 0, 0)),
            pl.BlockSpec((1, 1, BM), lambda b, k, be: (b, 0, 0)),
        ],
        out_specs=pl.BlockSpec((BM, D), lambda b, k, be: (b, 0)),
        scratch_shapes=[pltpu.VMEM((BM, D), jnp.float32)],
    )
    ys = pl.pallas_call(
        functools.partial(_ffn_kernel, n_ff_blocks=KF),
        grid_spec=grid_spec,
        out_shape=jax.ShapeDtypeStruct((P, D), jnp.float32),
    )(block_expert, xs, W1, B1r, W2, B2r, sw3)

    pos0 = posn[0::2]
    pos1 = posn[1::2]
    out2d = jnp.take(ys, pos0, axis=0) + jnp.take(ys, pos1, axis=0)

    return out2d.reshape(B, x.shape[1], D), loss
